# baseline (speedup 1.0000x reference)
import functools

import jax
import jax.numpy as jnp
from jax import lax
from jax.experimental import pallas as pl
from jax.experimental.pallas import tpu as pltpu
from jax.experimental.pallas import tpu_sc as plsc

DIM = 64
HIST = 200
HIST_PAD = 208
N_GROUPS = HIST_PAD // 16


def _body(idx_hbm, ui_hbm, item_hbm, user_hbm, w1_hbm, b1_hbm, w2_hbm,
          b2_hbm, out_hbm,
          idxall_v, ui_v, tiles_v, urow_v, w1_v, b1_v, w2_v, b2_v,
          mylog_v, alllog_v, c16_v, out_v, sh_log, sh_const, sem):
    cid = lax.axis_index("c")
    sid = lax.axis_index("s")
    is0 = cid == 0
    lanes = lax.iota(jnp.int32, 16)
    f32 = jnp.float32

    @pl.when(is0 & (sid < N_GROUPS))
    def _():
        pltpu.sync_copy(idx_hbm, idxall_v)
        idxc = idxall_v[pl.ds(sid * 16, 16)]
        descs = []
        for l in range(16):
            t8 = (idxc[l] // 8) * 8
            descs.append(pltpu.async_copy(
                item_hbm.at[pl.ds(t8, 8)],
                tiles_v.at[pl.ds(l * 8, 8)], sem))
        pltpu.sync_copy(w1_hbm, w1_v)
        pltpu.sync_copy(w2_hbm, w2_v)
        w2c = [w2_v[pl.ds(c * 16, 16)] for c in range(4)]
        vrow = [(DIM + c * 16 + lanes) // 2 for c in range(4)]
        vcol0 = (lanes % 2) * DIM
        vch = [jnp.zeros((16,), f32) for _ in range(4)]
        for k in range(DIM):
            w2b = jnp.full((16,), w2c[k // 16][k % 16], f32)
            colk = vcol0 + k
            for c in range(4):
                vch[c] = vch[c] + plsc.load_gather(w1_v, [vrow[c], colk]) * w2b
        for d in descs:
            d.wait()
        rowsel = lanes * 8 + idxc % 8
        acc = jnp.zeros((16,), f32)
        for d in range(DIM):
            vb = jnp.full((16,), vch[d // 16][d % 16], f32)
            acc = acc + plsc.load_gather(tiles_v, [rowsel, jnp.full((16,), d, jnp.int32)]) * vb
        mylog_v[...] = acc
        pltpu.sync_copy(mylog_v, sh_log.at[pl.ds(sid * 16, 16)])

    @pl.when(is0 & (sid == N_GROUPS))
    def _():
        pltpu.sync_copy(ui_hbm, ui_v)
        ui = ui_v[...][0]
        ud = pltpu.async_copy(user_hbm.at[pl.ds((ui // 8) * 8, 8)],
                              urow_v, sem)
        pltpu.sync_copy(w1_hbm, w1_v)
        pltpu.sync_copy(w2_hbm, w2_v)
        pltpu.sync_copy(b1_hbm, b1_v)
        pltpu.sync_copy(b2_hbm, b2_v)
        w2c = [w2_v[pl.ds(c * 16, 16)] for c in range(4)]
        vrow = [(c * 16 + lanes) // 2 for c in range(4)]
        vcol0 = (lanes % 2) * DIM
        vlo = [jnp.zeros((16,), f32) for _ in range(4)]
        for k in range(DIM):
            w2b = jnp.full((16,), w2c[k // 16][k % 16], f32)
            colk = vcol0 + k
            for c in range(4):
                vlo[c] = vlo[c] + plsc.load_gather(w1_v, [vrow[c], colk]) * w2b
        ud.wait()
        us = jnp.full((16,), ui % 8, jnp.int32)
        cvec = jnp.zeros((16,), f32)
        for c in range(4):
            uc = plsc.load_gather(urow_v, [us, c * 16 + lanes])
            cvec = cvec + uc * vlo[c] + b1_v[pl.ds(c * 16, 16)] * w2c[c]
        cconst = jnp.sum(cvec) + b2_v[...][0]
        c16_v[...] = jnp.full((16,), cconst, f32)
        pltpu.sync_copy(c16_v, sh_const)

    plsc.subcore_barrier()

    @pl.when(is0 & (sid == 0))
    def _():
        pltpu.sync_copy(sh_log, alllog_v)
        pltpu.sync_copy(sh_const, c16_v)
        cc = c16_v[...][0]
        logits = []
        for g in range(N_GROUPS):
            l = alllog_v[pl.ds(g * 16, 16)] + cc
            l = jnp.where(l >= 0.0, l, 0.01 * l)
            if (g + 1) * 16 > HIST:
                l = jnp.where(lanes + g * 16 < HIST, l, -1e30)
            logits.append(l)
        mvec = logits[0]
        for g in range(1, N_GROUPS):
            mvec = jnp.maximum(mvec, logits[g])
        m = jnp.max(mvec)
        exps = [jnp.exp(l - m) for l in logits]
        svec = exps[0]
        for g in range(1, N_GROUPS):
            svec = svec + exps[g]
        sb = jnp.full((16,), jnp.sum(svec), f32)
        inv = jnp.ones((16,), f32) / sb
        for g in range(N_GROUPS):
            out_v[pl.ds(g * 16, 16)] = exps[g] * inv
        pltpu.sync_copy(out_v.at[pl.ds(0, HIST)], out_hbm)


_sc_kernel = functools.partial(
    pl.kernel,
    out_type=jax.ShapeDtypeStruct((HIST,), jnp.float32),
    mesh=plsc.VectorSubcoreMesh(core_axis_name="c", subcore_axis_name="s"),
    compiler_params=pltpu.CompilerParams(needs_layout_passes=False,
                                         use_tc_tiling_on_sc=True),
    scratch_types=[
        pltpu.VMEM((HIST_PAD,), jnp.int32),
        pltpu.VMEM((16,), jnp.int32),
        pltpu.VMEM((16 * 8, DIM), jnp.float32),
        pltpu.VMEM((8, DIM), jnp.float32),
        pltpu.VMEM((DIM, 2 * DIM), jnp.float32),
        pltpu.VMEM((DIM,), jnp.float32),
        pltpu.VMEM((DIM,), jnp.float32),
        pltpu.VMEM((16,), jnp.float32),
        pltpu.VMEM((16,), jnp.float32),
        pltpu.VMEM((HIST_PAD,), jnp.float32),
        pltpu.VMEM((16,), jnp.float32),
        pltpu.VMEM((HIST_PAD,), jnp.float32),
        pltpu.VMEM_SHARED((HIST_PAD,), jnp.float32),
        pltpu.VMEM_SHARED((16,), jnp.float32),
        pltpu.SemaphoreType.DMA,
    ],
)(_body)


def kernel(user_indice, interacted_item_indices, user_table, item_table,
           W1, b1, W2, b2):
    idx = jnp.concatenate([
        interacted_item_indices.astype(jnp.int32),
        jnp.zeros((HIST_PAD - HIST,), jnp.int32),
    ])
    ui16 = jnp.full((16,), user_indice.astype(jnp.int32))
    w1r = W1.reshape(DIM, 2 * DIM)
    w2 = W2.reshape(DIM)
    b2p = jnp.concatenate([b2, jnp.zeros((15,), jnp.float32)])
    return _sc_kernel(idx, ui16, item_table, user_table, w1r, b1, w2, b2p)
